# Initial kernel scaffold; baseline (speedup 1.0000x reference)
import jax
import jax.numpy as jnp
from jax.experimental import pallas as pl


def kernel(x, W):
    raise NotImplementedError("write your pallas kernel here")


# speedup vs baseline: 5.3488x; 5.3488x over previous
import jax
import jax.numpy as jnp
from jax.experimental import pallas as pl
from jax.experimental.pallas import tpu as pltpu

_CHANNEL_IN = 256
_CHANNEL_OUT = 32768
_GROUP = 8
_BATCH = 128
_N_BLK = 2048


def _body(x_ref, w_ref, o_ref):
    acc = jnp.dot(x_ref[...], w_ref[...], preferred_element_type=jnp.float32)
    b, n = acc.shape
    r = acc.reshape(b, n // _GROUP, _GROUP)
    m = jnp.max(r, axis=2, keepdims=True)
    eq = r >= m
    iota = jax.lax.broadcasted_iota(jnp.int32, r.shape, 2)
    first = jnp.min(jnp.where(eq, iota, _GROUP), axis=2, keepdims=True)
    oh = (iota == first).astype(jnp.float32)
    o_ref[...] = oh.reshape(b, n)


@jax.jit
def kernel(x, W):
    return pl.pallas_call(
        _body,
        grid=(_CHANNEL_OUT // _N_BLK,),
        in_specs=[
            pl.BlockSpec((_BATCH, _CHANNEL_IN), lambda i: (0, 0)),
            pl.BlockSpec((_CHANNEL_IN, _N_BLK), lambda i: (0, i)),
        ],
        out_specs=pl.BlockSpec((_BATCH, _N_BLK), lambda i: (0, i)),
        out_shape=jax.ShapeDtypeStruct((_BATCH, _CHANNEL_OUT), jnp.float32),
        compiler_params=pltpu.CompilerParams(
            dimension_semantics=("parallel",),
        ),
    )(x, W)


# speedup vs baseline: 68.1528x; 12.7417x over previous
import jax
import jax.numpy as jnp
from jax.experimental import pallas as pl
from jax.experimental.pallas import tpu as pltpu

_CHANNEL_IN = 256
_CHANNEL_OUT = 32768
_GROUP = 8
_BATCH = 128
_N_BLK = 2048


def _body(x_ref, w_ref, o_ref):
    yt = jax.lax.dot_general(
        w_ref[...], x_ref[...],
        dimension_numbers=(((0,), (1,)), ((), ())),
        preferred_element_type=jnp.float32,
    )
    n, b = yt.shape
    r = yt.reshape(n // _GROUP, _GROUP, b)
    m = jnp.max(r, axis=1, keepdims=True)
    eq = r >= m
    iota = jax.lax.broadcasted_iota(jnp.int32, r.shape, 1)
    first = jnp.min(jnp.where(eq, iota, _GROUP), axis=1, keepdims=True)
    oh = (iota == first).astype(jnp.float32).reshape(n, b)
    o_ref[...] = oh.T


@jax.jit
def kernel(x, W):
    return pl.pallas_call(
        _body,
        grid=(_CHANNEL_OUT // _N_BLK,),
        in_specs=[
            pl.BlockSpec((_BATCH, _CHANNEL_IN), lambda i: (0, 0)),
            pl.BlockSpec((_CHANNEL_IN, _N_BLK), lambda i: (0, i)),
        ],
        out_specs=pl.BlockSpec((_BATCH, _N_BLK), lambda i: (0, i)),
        out_shape=jax.ShapeDtypeStruct((_BATCH, _CHANNEL_OUT), jnp.float32),
        compiler_params=pltpu.CompilerParams(
            dimension_semantics=("parallel",),
        ),
    )(x, W)
